# baseline (speedup 1.0000x reference)
import functools

import jax
import jax.numpy as jnp
from jax import lax
from jax.experimental import pallas as pl
from jax.experimental.pallas import tpu as pltpu
from jax.experimental.pallas import tpu_sc as plsc

_BETA = 0.25
_TM = 128


def _dist_argmin_body(x_ref, w_ref, idx_ref, loss_ref, acc_ref):
    i = pl.program_id(0)
    x = x_ref[...]
    w = w_ref[...]
    v = w.shape[0]
    rown = jnp.sum(x * x, axis=1, keepdims=True)
    coln = jnp.sum(w * w, axis=1)
    scores = lax.dot_general(
        x, w, (((1,), (1,)), ((), ())),
        preferred_element_type=jnp.float32)
    d = (rown + coln[None, :]) - 2.0 * scores
    dmin = jnp.min(d, axis=1)
    col = lax.broadcasted_iota(jnp.int32, d.shape, 1)
    idx = jnp.min(jnp.where(d == dmin[:, None], col, v), axis=1)
    idx_ref[...] = idx

    @pl.when(i == 0)
    def _():
        acc_ref[0] = 0.0

    acc_ref[0] += jnp.sum(dmin)
    n_total = pl.num_programs(0) * x.shape[0] * x.shape[1]
    loss_ref[...] = (acc_ref[0] * ((1.0 + _BETA) / n_total))[None, None]


def _dist_argmin(flat, w):
    n, d = flat.shape
    v = w.shape[0]
    grid = n // _TM
    return pl.pallas_call(
        _dist_argmin_body,
        grid=(grid,),
        in_specs=[
            pl.BlockSpec((_TM, d), lambda i: (i, 0)),
            pl.BlockSpec((v, d), lambda i: (0, 0)),
        ],
        out_specs=[
            pl.BlockSpec((_TM,), lambda i: (i,)),
            pl.BlockSpec((1, 1), lambda i: (0, 0)),
        ],
        out_shape=[
            jax.ShapeDtypeStruct((n,), jnp.int32),
            jax.ShapeDtypeStruct((1, 1), jnp.float32),
        ],
        scratch_shapes=[pltpu.SMEM((1,), jnp.float32)],
        compiler_params=pltpu.CompilerParams(
            dimension_semantics=("arbitrary",)),
    )(flat, w)


_NW = 32
_CH = 128


def _sc_gather(w, idx):
    n = idx.shape[0]
    d = w.shape[1]
    b_per_w = n // _NW
    mesh = plsc.VectorSubcoreMesh(core_axis_name="c", subcore_axis_name="s")

    @functools.partial(
        pl.kernel, mesh=mesh,
        out_type=jax.ShapeDtypeStruct((n, d), jnp.float32),
        scratch_types=[
            pltpu.VMEM((_CH,), jnp.int32),
            pltpu.VMEM((_CH, d), jnp.float32),
            pltpu.SemaphoreType.DMA,
        ],
    )
    def gather_k(table_hbm, idx_hbm, out_hbm, idx_v, rows_v, sem):
        wid = lax.axis_index("s") * 2 + lax.axis_index("c")
        base = wid * b_per_w

        def body(c, carry):
            off = base + c * _CH
            pltpu.sync_copy(idx_hbm.at[pl.ds(off, _CH)], idx_v)
            pltpu.async_copy(table_hbm.at[idx_v], rows_v, sem).wait()
            pltpu.sync_copy(rows_v, out_hbm.at[pl.ds(off, _CH)])
            return carry

        lax.fori_loop(0, b_per_w // _CH, body, 0)

    return gather_k(w, idx)


def kernel(inputs, W):
    shape = inputs.shape
    d = shape[-1]
    flat = inputs.reshape(-1, d)
    idx, loss = _dist_argmin(flat, W)
    z_q = _sc_gather(W, idx)
    return z_q.reshape(shape), loss[0, 0], idx
